# Initial kernel scaffold; baseline (speedup 1.0000x reference)
import functools

import jax
import jax.numpy as jnp
from jax import lax
from jax.experimental import pallas as pl
from jax.experimental.pallas import tpu as pltpu
from jax.experimental.pallas import tpu_sc as plsc

_N = 10000
_E = 320000
_L = 16

_NTILES = 32
_NPT = 313
_NPAD = _NTILES * _NPT
_CH = 2000
_NCHUNK = _E // _CH


def _make_segmax(D):
    mesh = plsc.VectorSubcoreMesh(core_axis_name="c", subcore_axis_name="s")

    @functools.partial(
        pl.kernel,
        mesh=mesh,
        out_type=jax.ShapeDtypeStruct((_NPAD, D), jnp.float32),
        scratch_types=[
            pltpu.VMEM((_CH,), jnp.int32),
            pltpu.VMEM((_CH,), jnp.int32),
            pltpu.VMEM((_CH,), jnp.float32),
            pltpu.VMEM((_CH + _L,), jnp.int32),
            pltpu.VMEM((_CH + _L,), jnp.int32),
            pltpu.VMEM((_CH + _L,), jnp.float32),
            pltpu.VMEM((_L, D), jnp.float32),
            pltpu.VMEM((_NPT + 1, D), jnp.float32),
            pltpu.SemaphoreType.DMA,
        ],
    )
    def segmax(feat, srcg, dstg, wg, out, dstc, srcc, wc, cdl, csr, cwt, rows, acc, sem):
        wid = lax.axis_index("s") * 2 + lax.axis_index("c")
        lo = wid * _NPT
        neg = jnp.float32(-jnp.inf)

        def init_row(r, carry):
            for k in range(D // _L):
                acc[r, pl.ds(k * _L, _L)] = jnp.full((_L,), neg, jnp.float32)
            return carry

        lax.fori_loop(0, _NPT + 1, init_row, 0)

        def chunk_body(ci, carry):
            base = ci * _CH
            pltpu.sync_copy(dstg.at[pl.ds(base, _CH)], dstc)
            pltpu.sync_copy(srcg.at[pl.ds(base, _CH)], srcc)
            pltpu.sync_copy(wg.at[pl.ds(base, _CH)], wc)

            def filt(i, ptr):
                dv = dstc[pl.ds(i * _L, _L)]
                sv = srcc[pl.ds(i * _L, _L)]
                wv = wc[pl.ds(i * _L, _L)]
                m = (dv >= lo) & (dv < lo + _NPT)
                mi = m.astype(jnp.int32)
                pos = ptr + jnp.cumsum(mi) - 1
                plsc.store_scatter(cdl, [pos], dv - lo, mask=m)
                plsc.store_scatter(csr, [pos], sv, mask=m)
                plsc.store_scatter(cwt, [pos], wv, mask=m)
                return ptr + jnp.sum(mi)

            cnt = lax.fori_loop(0, _CH // _L, filt, jnp.int32(0))

            padpos = cnt + lax.broadcasted_iota(jnp.int32, (_L,), 0)
            plsc.store_scatter(cdl, [padpos], jnp.full((_L,), _NPT, jnp.int32))
            plsc.store_scatter(csr, [padpos], jnp.zeros((_L,), jnp.int32))
            plsc.store_scatter(cwt, [padpos], jnp.zeros((_L,), jnp.float32))

            ng = (cnt + (_L - 1)) // _L

            def grp(j, c2):
                idxv = csr[pl.ds(j * _L, _L)]
                pltpu.async_copy(feat.at[idxv], rows, sem).wait()

                def lane(l, c3):
                    e = j * _L + l
                    dl = cdl[e]
                    wl = cwt[e]
                    for k in range(D // _L):
                        sl = pl.ds(k * _L, _L)
                        acc[dl, sl] = jnp.maximum(acc[dl, sl], rows[l, sl] * wl)
                    return c3

                lax.fori_loop(0, _L, lane, 0)
                return c2

            lax.fori_loop(0, ng, grp, 0)
            return carry

        lax.fori_loop(0, _NCHUNK, chunk_body, 0)

        def fin(r, carry):
            for k in range(D // _L):
                sl = pl.ds(k * _L, _L)
                v = acc[r, sl]
                acc[r, sl] = jnp.where(v == neg, jnp.float32(0.0), v)
            return carry

        lax.fori_loop(0, _NPT, fin, 0)

        pltpu.sync_copy(acc.at[pl.ds(0, _NPT)], out.at[pl.ds(lo, _NPT)])

    return segmax


_segmax128 = _make_segmax(128)
_segmax256 = _make_segmax(256)

_BR = 1000


def _tc1_body(agg_ref, x_ref, wr_ref, b_ref, wt_ref, o_ref):
    h = (jnp.dot(agg_ref[...], wr_ref[...], preferred_element_type=jnp.float32)
         + jnp.dot(x_ref[...], wt_ref[...], preferred_element_type=jnp.float32)
         + b_ref[...])
    o_ref[...] = jnp.maximum(h, 0.0)


def _tc1(agg, x, wrT, b, wtT):
    DIN, DH = wrT.shape
    return pl.pallas_call(
        _tc1_body,
        grid=(_N // _BR,),
        in_specs=[
            pl.BlockSpec((_BR, DIN), lambda i: (i, 0)),
            pl.BlockSpec((_BR, DIN), lambda i: (i, 0)),
            pl.BlockSpec((DIN, DH), lambda i: (0, 0)),
            pl.BlockSpec((1, DH), lambda i: (0, 0)),
            pl.BlockSpec((DIN, DH), lambda i: (0, 0)),
        ],
        out_specs=pl.BlockSpec((_BR, DH), lambda i: (i, 0)),
        out_shape=jax.ShapeDtypeStruct((_N, DH), jnp.float32),
    )(agg, x, wrT, b.reshape(1, DH), wtT)


def _tc2_body(agg_ref, h_ref, wr_ref, b_ref, wt_ref, wl_ref, bl_ref, o_ref):
    h = (jnp.dot(agg_ref[...], wr_ref[...], preferred_element_type=jnp.float32)
         + jnp.dot(h_ref[...], wt_ref[...], preferred_element_type=jnp.float32)
         + b_ref[...])
    h = jnp.maximum(h, 0.0)
    o_ref[...] = (jnp.dot(h, wl_ref[...], preferred_element_type=jnp.float32)
                  + bl_ref[...])


def _tc2(agg, h1, wrT, b, wtT, wlT, bl):
    DH, DOUT = wlT.shape
    return pl.pallas_call(
        _tc2_body,
        grid=(_N // _BR,),
        in_specs=[
            pl.BlockSpec((_BR, DH), lambda i: (i, 0)),
            pl.BlockSpec((_BR, DH), lambda i: (i, 0)),
            pl.BlockSpec((DH, DH), lambda i: (0, 0)),
            pl.BlockSpec((1, DH), lambda i: (0, 0)),
            pl.BlockSpec((DH, DH), lambda i: (0, 0)),
            pl.BlockSpec((DH, DOUT), lambda i: (0, 0)),
            pl.BlockSpec((1, DOUT), lambda i: (0, 0)),
        ],
        out_specs=pl.BlockSpec((_BR, DOUT), lambda i: (i, 0)),
        out_shape=jax.ShapeDtypeStruct((_N, DOUT), jnp.float32),
    )(agg, h1, wrT, b.reshape(1, DH), wtT, wlT, bl.reshape(1, DOUT))


def kernel(x, edge_index, edge_attr, W1_rel, b1_rel, W1_root, W2_rel, b2_rel, W2_root, W_lin, b_lin):
    src = edge_index[0]
    dst = edge_index[1]
    agg1 = _segmax128(x, src, dst, edge_attr)[:_N]
    h1 = _tc1(agg1, x, W1_rel.T, b1_rel, W1_root.T)
    agg2 = _segmax256(h1, src, dst, edge_attr)[:_N]
    out = _tc2(agg2, h1, W2_rel.T, b2_rel, W2_root.T, W_lin.T, b_lin)
    return out


# speedup vs baseline: 1.0203x; 1.0203x over previous
import functools

import jax
import jax.numpy as jnp
from jax import lax
from jax.experimental import pallas as pl
from jax.experimental.pallas import tpu as pltpu
from jax.experimental.pallas import tpu_sc as plsc

_N = 10000
_E = 320000
_L = 16

_NTILES = 32
_NPT = 320
_NPAD = _NTILES * _NPT
_CH = 2000
_NCHUNK = _E // _CH


@functools.lru_cache(maxsize=None)
def _make_segmax(D):
    mesh = plsc.VectorSubcoreMesh(core_axis_name="c", subcore_axis_name="s")

    @functools.partial(
        pl.kernel,
        mesh=mesh,
        compiler_params=pltpu.CompilerParams(needs_layout_passes=False),
        out_type=jax.ShapeDtypeStruct((_NPAD, D), jnp.float32),
        scratch_types=[
            pltpu.VMEM((_CH,), jnp.int32),
            pltpu.VMEM((_CH,), jnp.int32),
            pltpu.VMEM((_CH,), jnp.float32),
            pltpu.VMEM((_CH + _L,), jnp.int32),
            pltpu.VMEM((_CH + _L,), jnp.int32),
            pltpu.VMEM((_CH + _L,), jnp.float32),
            pltpu.VMEM((_L, D), jnp.float32),
            pltpu.VMEM((_NPT + 1, D), jnp.float32),
            pltpu.SemaphoreType.DMA,
        ],
    )
    def segmax(feat, srcg, dstg, wg, out, dstc, srcc, wc, cdl, csr, cwt, rows, acc, sem):
        wid = lax.axis_index("s") * 2 + lax.axis_index("c")
        lo = wid * _NPT
        neg = jnp.float32(-jnp.inf)

        def init_row(r, carry):
            for k in range(D // _L):
                acc[r, pl.ds(k * _L, _L)] = jnp.full((_L,), neg, jnp.float32)
            return carry

        lax.fori_loop(0, _NPT + 1, init_row, 0)

        def chunk_body(ci, carry):
            base = ci * _CH
            pltpu.sync_copy(dstg.at[pl.ds(base, _CH)], dstc)
            pltpu.sync_copy(srcg.at[pl.ds(base, _CH)], srcc)
            pltpu.sync_copy(wg.at[pl.ds(base, _CH)], wc)

            def filt(i, ptr):
                dv = dstc[pl.ds(i * _L, _L)]
                sv = srcc[pl.ds(i * _L, _L)]
                wv = wc[pl.ds(i * _L, _L)]
                m = (dv >= lo) & (dv < lo + _NPT)
                mi = m.astype(jnp.int32)
                pos = ptr + jnp.cumsum(mi) - 1
                plsc.store_scatter(cdl, [pos], dv - lo, mask=m)
                plsc.store_scatter(csr, [pos], sv, mask=m)
                plsc.store_scatter(cwt, [pos], wv, mask=m)
                return ptr + jnp.sum(mi)

            cnt = lax.fori_loop(0, _CH // _L, filt, jnp.int32(0))

            padpos = cnt + lax.broadcasted_iota(jnp.int32, (_L,), 0)
            plsc.store_scatter(cdl, [padpos], jnp.full((_L,), _NPT, jnp.int32))
            plsc.store_scatter(csr, [padpos], jnp.zeros((_L,), jnp.int32))
            plsc.store_scatter(cwt, [padpos], jnp.zeros((_L,), jnp.float32))

            ng = (cnt + (_L - 1)) // _L

            def grp(j, c2):
                idxv = csr[pl.ds(j * _L, _L)]
                pltpu.async_copy(feat.at[idxv], rows, sem).wait()
                dlv = cdl[pl.ds(j * _L, _L)]
                wlv = cwt[pl.ds(j * _L, _L)]
                for l in range(_L):
                    dl = dlv[l]
                    wl = wlv[l]
                    for k in range(D // _L):
                        sl = pl.ds(k * _L, _L)
                        acc[dl, sl] = jnp.maximum(acc[dl, sl], rows[l, sl] * wl)
                return c2

            lax.fori_loop(0, ng, grp, 0)
            return carry

        lax.fori_loop(0, _NCHUNK, chunk_body, 0)

        def fin(r, carry):
            for k in range(D // _L):
                sl = pl.ds(k * _L, _L)
                v = acc[r, sl]
                acc[r, sl] = jnp.where(v == neg, jnp.float32(0.0), v)
            return carry

        lax.fori_loop(0, _NPT, fin, 0)

        pltpu.sync_copy(acc.at[pl.ds(0, _NPT)], out.at[pl.ds(lo, _NPT)])

    return segmax


_BR = 1000


def _tc1_body(agg_ref, x_ref, wr_ref, b_ref, wt_ref, o_ref):
    h = (jnp.dot(agg_ref[...], wr_ref[...], preferred_element_type=jnp.float32)
         + jnp.dot(x_ref[...], wt_ref[...], preferred_element_type=jnp.float32)
         + b_ref[...])
    o_ref[...] = jnp.maximum(h, 0.0)


def _tc1(agg, x, wrT, b, wtT):
    DIN, DH = wrT.shape
    return pl.pallas_call(
        _tc1_body,
        grid=(_N // _BR,),
        in_specs=[
            pl.BlockSpec((_BR, DIN), lambda i: (i, 0)),
            pl.BlockSpec((_BR, DIN), lambda i: (i, 0)),
            pl.BlockSpec((DIN, DH), lambda i: (0, 0)),
            pl.BlockSpec((1, DH), lambda i: (0, 0)),
            pl.BlockSpec((DIN, DH), lambda i: (0, 0)),
        ],
        out_specs=pl.BlockSpec((_BR, DH), lambda i: (i, 0)),
        out_shape=jax.ShapeDtypeStruct((_N, DH), jnp.float32),
    )(agg, x, wrT, b.reshape(1, DH), wtT)


def _tc2_body(agg_ref, h_ref, wr_ref, b_ref, wt_ref, wl_ref, bl_ref, o_ref):
    h = (jnp.dot(agg_ref[...], wr_ref[...], preferred_element_type=jnp.float32)
         + jnp.dot(h_ref[...], wt_ref[...], preferred_element_type=jnp.float32)
         + b_ref[...])
    h = jnp.maximum(h, 0.0)
    o_ref[...] = (jnp.dot(h, wl_ref[...], preferred_element_type=jnp.float32)
                  + bl_ref[...])


def _tc2(agg, h1, wrT, b, wtT, wlT, bl):
    DH, DOUT = wlT.shape
    return pl.pallas_call(
        _tc2_body,
        grid=(_N // _BR,),
        in_specs=[
            pl.BlockSpec((_BR, DH), lambda i: (i, 0)),
            pl.BlockSpec((_BR, DH), lambda i: (i, 0)),
            pl.BlockSpec((DH, DH), lambda i: (0, 0)),
            pl.BlockSpec((1, DH), lambda i: (0, 0)),
            pl.BlockSpec((DH, DH), lambda i: (0, 0)),
            pl.BlockSpec((DH, DOUT), lambda i: (0, 0)),
            pl.BlockSpec((1, DOUT), lambda i: (0, 0)),
        ],
        out_specs=pl.BlockSpec((_BR, DOUT), lambda i: (i, 0)),
        out_shape=jax.ShapeDtypeStruct((_N, DOUT), jnp.float32),
    )(agg, h1, wrT, b.reshape(1, DH), wtT, wlT, bl.reshape(1, DOUT))


def kernel(x, edge_index, edge_attr, W1_rel, b1_rel, W1_root, W2_rel, b2_rel, W2_root, W_lin, b_lin):
    src = edge_index[0]
    dst = edge_index[1]
    agg1 = _make_segmax(128)(x, src, dst, edge_attr)[:_N]
    h1 = _tc1(agg1, x, W1_rel.T, b1_rel, W1_root.T)
    agg2 = _make_segmax(256)(h1, src, dst, edge_attr)[:_N]
    out = _tc2(agg2, h1, W2_rel.T, b2_rel, W2_root.T, W_lin.T, b_lin)
    return out
